# Initial kernel scaffold; baseline (speedup 1.0000x reference)
import jax
import jax.numpy as jnp
from jax.experimental import pallas as pl


def kernel(token_ids, weight):
    raise NotImplementedError("write your pallas kernel here")


# speedup vs baseline: 1.0950x; 1.0950x over previous
import functools

import jax
import jax.numpy as jnp
from jax import lax
from jax.experimental import pallas as pl
from jax.experimental.pallas import tpu as pltpu
from jax.experimental.pallas import tpu_sc as plsc

NUM_WORKERS = 32
GATHER_W = 128
CHUNK_GATHERS = 8
CHUNK = GATHER_W * CHUNK_GATHERS


def _make_kernel(B, D):
    per_w = B // NUM_WORKERS
    chunks_per_w = per_w // CHUNK
    idx_rows_per_w = per_w // GATHER_W
    mesh = plsc.VectorSubcoreMesh(core_axis_name="c", subcore_axis_name="s")

    @functools.partial(
        pl.kernel,
        out_type=jax.ShapeDtypeStruct((B, D), jnp.float32),
        mesh=mesh,
        scratch_types=[
            pltpu.VMEM((CHUNK_GATHERS, GATHER_W), jnp.int32),
            pltpu.VMEM((CHUNK, D), jnp.float32),
            pltpu.SemaphoreType.DMA,
        ],
        compiler_params=pltpu.CompilerParams(use_tc_tiling_on_sc=False),
    )
    def gather_kernel(idx_hbm, table_hbm, out_hbm, idx_v, rows_v, sem):
        wid = lax.axis_index("s") * 2 + lax.axis_index("c")
        idx_row0 = wid * idx_rows_per_w
        out0 = wid * per_w

        def body(g, carry):
            pltpu.sync_copy(
                idx_hbm.at[pl.ds(idx_row0 + g * CHUNK_GATHERS, CHUNK_GATHERS)],
                idx_v,
            )
            for j in range(CHUNK_GATHERS):
                pltpu.async_copy(
                    table_hbm.at[idx_v.at[j]],
                    rows_v.at[pl.ds(j * GATHER_W, GATHER_W)],
                    sem,
                )
            for j in range(CHUNK_GATHERS):
                pltpu.make_async_copy(
                    table_hbm.at[idx_v.at[j]],
                    rows_v.at[pl.ds(j * GATHER_W, GATHER_W)],
                    sem,
                ).wait()
            pltpu.sync_copy(rows_v, out_hbm.at[pl.ds(out0 + g * CHUNK, CHUNK)])
            return carry

        lax.fori_loop(0, chunks_per_w, body, 0)

    return gather_kernel


def kernel(token_ids, weight):
    S0, S1 = token_ids.shape
    B = S0 * S1
    D = weight.shape[1]
    idx = token_ids.reshape(B // GATHER_W, GATHER_W).astype(jnp.int32)
    out = _make_kernel(B, D)(idx, weight)
    return out.reshape(S0, S1, D)


# speedup vs baseline: 1.1111x; 1.0146x over previous
import functools

import jax
import jax.numpy as jnp
from jax import lax
from jax.experimental import pallas as pl
from jax.experimental.pallas import tpu as pltpu
from jax.experimental.pallas import tpu_sc as plsc

NUM_WORKERS = 32
GATHER_W = 128
CHUNK_GATHERS = 10
CHUNK = GATHER_W * CHUNK_GATHERS


def _make_kernel(B, D):
    per_w = B // NUM_WORKERS
    chunks_per_w = per_w // CHUNK
    idx_rows_per_w = per_w // GATHER_W
    mesh = plsc.VectorSubcoreMesh(core_axis_name="c", subcore_axis_name="s")

    @functools.partial(
        pl.kernel,
        out_type=jax.ShapeDtypeStruct((B, D), jnp.float32),
        mesh=mesh,
        scratch_types=[
            pltpu.VMEM((idx_rows_per_w, GATHER_W), jnp.int32),
            pltpu.VMEM((2, CHUNK, D), jnp.float32),
            pltpu.SemaphoreType.DMA,
            pltpu.SemaphoreType.DMA,
            pltpu.SemaphoreType.DMA,
        ],
        compiler_params=pltpu.CompilerParams(use_tc_tiling_on_sc=False),
    )
    def gather_kernel(idx_hbm, table_hbm, out_hbm, idx_v, rows_v, sem_g,
                      sem_w0, sem_w1):
        wid = lax.axis_index("s") * 2 + lax.axis_index("c")
        idx_row0 = wid * idx_rows_per_w
        out0 = wid * per_w
        sem_w = (sem_w0, sem_w1)

        pltpu.sync_copy(idx_hbm.at[pl.ds(idx_row0, idx_rows_per_w)], idx_v)

        def fire_gathers(c, b):
            for j in range(CHUNK_GATHERS):
                pltpu.async_copy(
                    table_hbm.at[idx_v.at[c * CHUNK_GATHERS + j]],
                    rows_v.at[b, pl.ds(j * GATHER_W, GATHER_W)],
                    sem_g,
                )

        def drain_gathers(b):
            for j in range(CHUNK_GATHERS):
                pltpu.make_async_copy(
                    table_hbm.at[idx_v.at[j]],
                    rows_v.at[b, pl.ds(j * GATHER_W, GATHER_W)],
                    sem_g,
                ).wait()

        def writeback(c, b):
            return pltpu.make_async_copy(
                rows_v.at[b], out_hbm.at[pl.ds(out0 + c * CHUNK, CHUNK)],
                sem_w[b],
            )

        fire_gathers(0, 0)
        drain_gathers(0)
        writeback(0, 0).start()
        fire_gathers(1, 1)
        drain_gathers(1)
        writeback(1, 1).start()

        def body(g, carry):
            for b in range(2):
                c = 2 * g + b
                writeback(c - 2, b).wait()
                fire_gathers(c, b)
                drain_gathers(b)
                writeback(c, b).start()
            return carry

        lax.fori_loop(1, chunks_per_w // 2, body, 0)

        writeback(chunks_per_w - 2, 0).wait()
        writeback(chunks_per_w - 1, 1).wait()

    return gather_kernel


def kernel(token_ids, weight):
    S0, S1 = token_ids.shape
    B = S0 * S1
    D = weight.shape[1]
    idx = token_ids.reshape(B // GATHER_W, GATHER_W).astype(jnp.int32)
    out = _make_kernel(B, D)(idx, weight)
    return out.reshape(S0, S1, D)


# speedup vs baseline: 1.6381x; 1.4743x over previous
import functools

import jax
import jax.numpy as jnp
from jax import lax
from jax.experimental import pallas as pl
from jax.experimental.pallas import tpu as pltpu
from jax.experimental.pallas import tpu_sc as plsc

NUM_WORKERS = 32
L = 128
NB = 4
NS = 50


def _make_kernel(D):
    mesh = plsc.VectorSubcoreMesh(core_axis_name="c", subcore_axis_name="s")

    @functools.partial(
        pl.kernel,
        out_type=jax.ShapeDtypeStruct((NS, 128, L, D), jnp.float32),
        mesh=mesh,
        scratch_types=[
            pltpu.VMEM((NS + 2, NB, L), jnp.int32),
            pltpu.VMEM((2, NB, L, D), jnp.float32),
            pltpu.SemaphoreType.DMA,
            pltpu.SemaphoreType.DMA,
            pltpu.SemaphoreType.DMA,
        ],
        compiler_params=pltpu.CompilerParams(use_tc_tiling_on_sc=False),
    )
    def gather_kernel(idx_hbm, table_hbm, out_hbm, idx_v, rows_v,
                      sem_g, sem_w0, sem_w1):
        wid = lax.axis_index("s") * 2 + lax.axis_index("c")
        bb0 = wid * NB
        sem_w = (sem_w0, sem_w1)

        pltpu.sync_copy(idx_hbm.at[:, pl.ds(bb0, NB)],
                        idx_v.at[pl.ds(0, NS)])
        zeros16 = jnp.zeros((16,), jnp.int32)
        for r in range(NS, NS + 2):
            for g in range(NB):
                for k in range(L // 16):
                    idx_v[r, g, pl.ds(k * 16, 16)] = zeros16

        def fire_gathers(s, par):
            for g in range(NB):
                pltpu.async_copy(
                    table_hbm.at[idx_v.at[s, g]],
                    rows_v.at[par, g],
                    sem_g,
                )

        def drain_gathers(par):
            for g in range(NB):
                pltpu.make_async_copy(
                    table_hbm.at[idx_v.at[0, g]],
                    rows_v.at[par, g],
                    sem_g,
                ).wait()

        def writeback(s, par):
            return pltpu.make_async_copy(
                rows_v.at[par],
                out_hbm.at[s, pl.ds(bb0, NB)],
                sem_w[par],
            )

        fire_gathers(0, 0)
        drain_gathers(0)
        fire_gathers(1, 1)
        writeback(0, 0).start()
        drain_gathers(1)
        writeback(0, 0).wait()
        fire_gathers(2, 0)
        writeback(1, 1).start()

        def body(p, carry):
            for par in range(2):
                s = 2 * p + par
                drain_gathers(par)
                writeback(s - 1, 1 - par).wait()
                fire_gathers(s + 1, 1 - par)
                writeback(s, par).start()
            return carry

        lax.fori_loop(1, NS // 2, body, 0)

        drain_gathers(0)
        writeback(NS - 1, 1).wait()

    return gather_kernel


def kernel(token_ids, weight):
    S0, S1 = token_ids.shape
    D = weight.shape[1]
    idx = token_ids.T.reshape(S1, S0 // L, L).astype(jnp.int32)
    out = _make_kernel(D)(idx, weight)
    return out.reshape(S1, S0, D).transpose(1, 0, 2)
